# baseline (speedup 1.0000x reference)
import functools

import jax
import jax.numpy as jnp
from jax import lax
from jax.experimental import pallas as pl
from jax.experimental.pallas import tpu as pltpu
from jax.experimental.pallas import tpu_sc as plsc

NC = 2
NS = 16
NW = NC * NS
CHUNK = 128
GROUP = 8


def _sc_mesh():
    return plsc.VectorSubcoreMesh(core_axis_name="c", subcore_axis_name="s")


def _make_scatter_kernel(n_pad, d, groups_per_tile):
    rpt = n_pad // NS

    @functools.partial(
        pl.kernel,
        out_type=jax.ShapeDtypeStruct((NC, n_pad, d), jnp.float32),
        mesh=_sc_mesh(),
        scratch_types=[
            pltpu.VMEM_SHARED((n_pad, d), jnp.float32),
            pltpu.VMEM((GROUP, CHUNK), jnp.int32),
            pltpu.VMEM((GROUP, CHUNK), jnp.int32),
            pltpu.VMEM((CHUNK, d), jnp.float32),
            pltpu.VMEM((CHUNK, d), jnp.float32),
            pltpu.SemaphoreType.DMA,
            pltpu.SemaphoreType.DMA,
        ],
    )
    def scatter_kernel(g_hbm, row_hbm, col_hbm, out_hbm,
                       acc, rowv, colv, rows0, rows1, sem0, sem1):
        cid = lax.axis_index("c")
        sid = lax.axis_index("s")
        wid = cid * NS + sid
        def zrow(r, carry):
            for k in range(d // 16):
                rows0[r, pl.ds(k * 16, 16)] = jnp.zeros((16,), jnp.float32)
            return carry

        lax.fori_loop(0, CHUNK, zrow, 0, unroll=False)
        for t in range(rpt // CHUNK):
            pltpu.sync_copy(
                rows0, acc.at[pl.ds(sid * rpt + t * CHUNK, CHUNK)])
        plsc.subcore_barrier()
        bufs = (rows0, rows1)
        sems = (sem0, sem1)

        def group_body(gidx, carry):
            pltpu.sync_copy(row_hbm.at[wid, pl.ds(gidx * GROUP, GROUP)], rowv)
            pltpu.sync_copy(col_hbm.at[wid, pl.ds(gidx * GROUP, GROUP)], colv)
            for j in range(GROUP):
                pltpu.async_copy(g_hbm.at[rowv.at[j]], bufs[j % 2],
                                 sems[j % 2]).wait()
                pltpu.sync_copy(bufs[j % 2], acc.at[colv.at[j]], add=True)
            return carry

        lax.fori_loop(0, groups_per_tile, group_body, 0, unroll=False)
        plsc.subcore_barrier()
        pltpu.sync_copy(acc.at[pl.ds(sid * rpt, rpt)],
                        out_hbm.at[cid, pl.ds(sid * rpt, rpt)])

    return scatter_kernel


def _mm_scale_body(dis_ref, x_ref, w_ref, o_ref):
    h = lax.dot_general(x_ref[...], w_ref[...], (((1,), (1,)), ((), ())),
                        preferred_element_type=jnp.float32)
    o_ref[...] = h * dis_ref[...]


def _layer2_body(dis_ref, p_ref, g1_ref, b1_ref, w2_ref, o_ref):
    dis = dis_ref[...]
    x1 = dis * (p_ref[0] + p_ref[1] + g1_ref[...]) + b1_ref[...]
    xr = jnp.maximum(x1, 0.0)
    h2 = lax.dot_general(xr, w2_ref[...], (((1,), (1,)), ((), ())),
                         preferred_element_type=jnp.float32)
    o_ref[...] = h2 * dis


def _final_body(dis_ref, q_ref, g2_ref, b2_ref, o_ref):
    o_ref[...] = (dis_ref[...] * (q_ref[0] + q_ref[1] + g2_ref[...])
                  + b2_ref[...])


def _tc_call(body, n, d, extra_specs, extra_args, row_block):
    grid = n // row_block
    dis_spec = pl.BlockSpec((row_block, 1), lambda i: (i, 0))
    return pl.pallas_call(
        body,
        grid=(grid,),
        in_specs=[dis_spec] + extra_specs,
        out_specs=pl.BlockSpec((row_block, d), lambda i: (i, 0)),
        out_shape=jax.ShapeDtypeStruct((n, d), jnp.float32),
    )(*extra_args)


def kernel(x, edge_index, W1, b1, W2, b2):
    n, d_in = x.shape
    d_hid = W1.shape[0]
    d_out = W2.shape[0]
    e = edge_index.shape[1]

    ei = edge_index.astype(jnp.int32)
    row, col = ei[0], ei[1]

    epw = NW * CHUNK * GROUP
    e_pad = ((e + epw - 1) // epw) * epw
    npw = NS * CHUNK
    n_pad = ((n + npw - 1) // npw) * npw
    if n_pad == n:
        n_pad = n + npw
    rpt = n_pad // NS

    pad = e_pad - e
    row_p = jnp.concatenate([row, jnp.zeros((pad,), jnp.int32)])
    col_p = jnp.concatenate([col, jnp.full((pad,), n, jnp.int32)])
    row_sc = row_p.reshape(NW, e_pad // (NW * CHUNK), CHUNK)
    col_sc = col_p.reshape(NW, e_pad // (NW * CHUNK), CHUNK)

    scatter = _make_scatter_kernel(n_pad, d_hid, e_pad // (NW * CHUNK * GROUP))

    ones_g = jnp.ones((n, d_hid), jnp.float32)
    deg_parts = scatter(ones_g, jnp.zeros_like(row_sc), col_sc)

    dis2d = lax.rsqrt(deg_parts[0, :n, 0] + deg_parts[1, :n, 0] + 1.0)[:, None]

    row_block = 1000 if n % 1000 == 0 else 8
    full_w = pl.BlockSpec((d_in, d_in), lambda i: (0, 0))
    xb = pl.BlockSpec((row_block, d_in), lambda i: (i, 0))
    bias_spec = pl.BlockSpec((1, d_hid), lambda i: (0, 0))
    part_spec = pl.BlockSpec((NC, row_block, d_hid), lambda i: (0, i, 0))
    gb = pl.BlockSpec((row_block, d_hid), lambda i: (i, 0))

    g1 = _tc_call(_mm_scale_body, n, d_hid,
                  [xb, full_w], [dis2d, x, W1], row_block)

    p = scatter(g1, row_sc, col_sc)

    g2 = _tc_call(_layer2_body, n, d_out,
                  [part_spec, gb, bias_spec, full_w],
                  [dis2d, p, g1, b1.reshape(1, -1), W2], row_block)

    q = scatter(g2, row_sc, col_sc)

    out = _tc_call(_final_body, n, d_out,
                   [part_spec, gb, bias_spec],
                   [dis2d, q, g2, b2.reshape(1, -1)], row_block)
    return out
